# Initial kernel scaffold; baseline (speedup 1.0000x reference)
import jax
import jax.numpy as jnp
from jax.experimental import pallas as pl


def kernel(x, edge_attr, W_node, b_node, W_edge, b_edge, W1, b1, W2, b2, W_out, b_out, edge_index, batch_vec):
    raise NotImplementedError("write your pallas kernel here")


# speedup vs baseline: 2.9871x; 2.9871x over previous
import functools

import jax
import jax.numpy as jnp
from jax import lax
from jax.experimental import pallas as pl
from jax.experimental.pallas import tpu as pltpu
from jax.experimental.pallas import tpu_sc as plsc

N = 10000
E = 320000
HID = 128
G = 128
L = 16
NC = 2
NS = 16
NW = NC * NS
EPT = E // NW
IW = 125
RPC = 1
CH = RPC * IW
NCHUNKS = EPT // CH
PADN = 10240
SLAB = PADN // NS


def _sc_message(h, eh3d, src3d, dst3d):
    mesh = plsc.VectorSubcoreMesh(core_axis_name="c", subcore_axis_name="s")

    @functools.partial(
        pl.kernel,
        out_type=jax.ShapeDtypeStruct((NC, PADN, HID), jnp.float32),
        mesh=mesh,
        scratch_types=[
            pltpu.VMEM((RPC, IW), jnp.int32),
            pltpu.VMEM((RPC, IW), jnp.int32),
            pltpu.VMEM((CH, HID), jnp.float32),
            pltpu.VMEM((CH, HID), jnp.float32),
            pltpu.VMEM_SHARED((PADN, HID), jnp.float32),
            pltpu.SemaphoreType.DMA,
        ],
    )
    def kfn(h_hbm, eh_hbm, src_hbm, dst_hbm, out_hbm, srcv, dstv, rowsv, ehv,
            agg, sem):
        c = lax.axis_index("c")
        s = lax.axis_index("s")
        w = c * NS + s

        zero = jnp.zeros((L,), jnp.float32)

        def zrow(r, carry):
            for q in range(HID // L):
                rowsv[r, pl.ds(q * L, L)] = zero
            return carry

        lax.fori_loop(0, CH, zrow, 0)

        base = pl.multiple_of(s * SLAB, 8)
        for off, nn in ((0, 120), (120, 120), (240, 120), (360, 120),
                        (480, 120), (600, 40)):
            pltpu.sync_copy(rowsv.at[pl.ds(0, nn)],
                            agg.at[pl.ds(pl.multiple_of(s * SLAB + off, 8),
                                         nn)])
        plsc.subcore_barrier()

        def chunk(k, carry):
            kk = w * NCHUNKS + k
            pltpu.sync_copy(src_hbm.at[kk], srcv)
            pltpu.sync_copy(dst_hbm.at[kk], dstv)
            pltpu.sync_copy(eh_hbm.at[kk], ehv)
            cps = [
                pltpu.async_copy(h_hbm.at[srcv.at[j]],
                                 rowsv.at[pl.ds(j * IW, IW)], sem)
                for j in range(RPC)
            ]
            for cp in cps:
                cp.wait()

            def erow(r, cc):
                for q in range(HID // L):
                    sl = pl.ds(q * L, L)
                    rowsv[r, sl] = jnp.maximum(rowsv[r, sl] + ehv[r, sl], 0.0)
                return cc

            lax.fori_loop(0, CH, erow, 0)
            for j in range(RPC):
                pltpu.sync_copy(rowsv.at[pl.ds(j * IW, IW)],
                                agg.at[dstv.at[j]], add=True)
            return carry

        lax.fori_loop(0, NCHUNKS, chunk, 0)
        plsc.subcore_barrier()
        pltpu.sync_copy(agg.at[pl.ds(base, SLAB)],
                        out_hbm.at[c, pl.ds(base, SLAB)])

    return kfn(h, eh3d, src3d, dst3d)


def _tc_matmul_bias(xm, wm, b2d, bm, relu):
    m, k = xm.shape
    n = wm.shape[1]

    def body(x_ref, w_ref, b_ref, o_ref):
        acc = jnp.dot(x_ref[...], w_ref[...],
                      preferred_element_type=jnp.float32) + b_ref[...]
        o_ref[...] = jnp.maximum(acc, 0.0) if relu else acc

    return pl.pallas_call(
        body,
        grid=(m // bm,),
        in_specs=[
            pl.BlockSpec((bm, k), lambda i: (i, 0)),
            pl.BlockSpec((k, n), lambda i: (0, 0)),
            pl.BlockSpec((1, n), lambda i: (0, 0)),
        ],
        out_specs=pl.BlockSpec((bm, n), lambda i: (i, 0)),
        out_shape=jax.ShapeDtypeStruct((m, n), jnp.float32),
    )(xm, wm, b2d)


def _tc_mlp(agg_p, w1, b1_2d, w2, b2_2d, bm):
    def body(a_ref, w1_ref, b1_ref, w2_ref, b2_ref, o_ref):
        sm = a_ref[0] + a_ref[1]
        z = jnp.maximum(
            jnp.dot(sm, w1_ref[...], preferred_element_type=jnp.float32)
            + b1_ref[...], 0.0)
        o_ref[...] = jnp.maximum(
            jnp.dot(z, w2_ref[...], preferred_element_type=jnp.float32)
            + b2_ref[...], 0.0)

    return pl.pallas_call(
        body,
        grid=(N // bm,),
        in_specs=[
            pl.BlockSpec((NC, bm, HID), lambda i: (0, i, 0)),
            pl.BlockSpec((HID, HID), lambda i: (0, 0)),
            pl.BlockSpec((1, HID), lambda i: (0, 0)),
            pl.BlockSpec((HID, HID), lambda i: (0, 0)),
            pl.BlockSpec((1, HID), lambda i: (0, 0)),
        ],
        out_specs=pl.BlockSpec((bm, HID), lambda i: (i, 0)),
        out_shape=jax.ShapeDtypeStruct((N, HID), jnp.float32),
    )(agg_p, w1, b1_2d, w2, b2_2d)


def _tc_readout(h, w_out, b_out_2d, bv3d, bm):
    nblk = N // bm

    def body(h_ref, w_ref, bo_ref, bv_ref, o_ref):
        i = pl.program_id(0)
        hv = jnp.dot(h_ref[...], w_ref[...],
                     preferred_element_type=jnp.float32,
                     precision=lax.Precision.HIGHEST)
        bv = bv_ref[0, 0, :]
        gi = lax.broadcasted_iota(jnp.int32, (bm, G), 1)
        oh = (bv[:, None] == gi).astype(jnp.float32)
        contrib = lax.dot_general(hv, oh, (((0,), (0,)), ((), ())),
                                  preferred_element_type=jnp.float32,
                                  precision=lax.Precision.HIGHEST)

        @pl.when(i == 0)
        def _():
            o_ref[...] = contrib + bo_ref[...]

        @pl.when(i != 0)
        def _():
            o_ref[...] = o_ref[...] + contrib

    return pl.pallas_call(
        body,
        grid=(nblk,),
        in_specs=[
            pl.BlockSpec((bm, HID), lambda i: (i, 0)),
            pl.BlockSpec((HID, 1), lambda i: (0, 0)),
            pl.BlockSpec((1, 1), lambda i: (0, 0)),
            pl.BlockSpec((1, 1, bm), lambda i: (i, 0, 0)),
        ],
        out_specs=pl.BlockSpec((1, G), lambda i: (0, 0)),
        out_shape=jax.ShapeDtypeStruct((1, G), jnp.float32),
    )(h, w_out, b_out_2d, bv3d)


def kernel(x, edge_attr, W_node, b_node, W_edge, b_edge, W1, b1, W2, b2,
           W_out, b_out, edge_index, batch_vec):
    h = _tc_matmul_bias(x, W_node, b_node.reshape(1, -1), bm=2000, relu=True)
    eh = _tc_matmul_bias(edge_attr, W_edge, b_edge.reshape(1, -1), bm=8000,
                         relu=False)
    eh3d = eh.reshape(E // CH, CH, HID)
    src3d = edge_index[0].reshape(E // CH, RPC, IW)
    dst3d = edge_index[1].reshape(E // CH, RPC, IW)
    for l in range(W1.shape[0]):
        agg_p = _sc_message(h, eh3d, src3d, dst3d)
        h = _tc_mlp(agg_p, W1[l], b1[l].reshape(1, -1), W2[l],
                    b2[l].reshape(1, -1), bm=2000)
    pred_t = _tc_readout(h, W_out, b_out.reshape(1, 1),
                         batch_vec.reshape(N // 2000, 1, 2000), bm=2000)
    return pred_t.reshape(G, 1)


# speedup vs baseline: 5.6722x; 1.8989x over previous
import functools

import jax
import jax.numpy as jnp
from jax import lax
from jax.experimental import pallas as pl
from jax.experimental.pallas import tpu as pltpu
from jax.experimental.pallas import tpu_sc as plsc

N = 10000
E = 320000
HID = 128
G = 128
L = 16
NC = 2
NS = 16
NW = NC * NS
EPT = E // NW
CH = 40
NCHUNKS = EPT // CH
SLOTS = 4
PADN = 10240
SLAB = PADN // NS


def _sc_message(h, eh3d, idx3d):
    mesh = plsc.VectorSubcoreMesh(core_axis_name="c", subcore_axis_name="s")

    @functools.partial(
        pl.kernel,
        out_type=jax.ShapeDtypeStruct((NC, PADN, HID), jnp.float32),
        mesh=mesh,
        scratch_types=(
            [pltpu.VMEM((2, CH), jnp.int32) for _ in range(SLOTS)]
            + [pltpu.VMEM((CH, HID), jnp.float32) for _ in range(SLOTS)]
            + [pltpu.VMEM((CH, HID), jnp.float32) for _ in range(SLOTS)]
            + [pltpu.SemaphoreType.DMA for _ in range(3 * SLOTS)]
            + [pltpu.VMEM_SHARED((PADN, HID), jnp.float32)]
        ),
    )
    def kfn(h_hbm, eh_hbm, idx_hbm, out_hbm, *refs):
        idxv = refs[0:SLOTS]
        rowsv = refs[SLOTS:2 * SLOTS]
        ehv = refs[2 * SLOTS:3 * SLOTS]
        isem = refs[3 * SLOTS:4 * SLOTS]
        gsem = refs[4 * SLOTS:5 * SLOTS]
        ssem = refs[5 * SLOTS:6 * SLOTS]
        agg = refs[6 * SLOTS]

        c = lax.axis_index("c")
        s = lax.axis_index("s")
        w = c * NS + s

        def issue_idx(k, j):
            pltpu.async_copy(idx_hbm.at[w * NCHUNKS + k], idxv[j], isem[j])

        def wait_idx(j):
            pltpu.make_async_copy(idx_hbm.at[0], idxv[j], isem[j]).wait()

        def issue_ge(k, j):
            pltpu.async_copy(h_hbm.at[idxv[j].at[0]], rowsv[j], gsem[j])
            pltpu.async_copy(eh_hbm.at[w * NCHUNKS + k], ehv[j], gsem[j])

        def wait_ge(j):
            pltpu.make_async_copy(h_hbm.at[idxv[j].at[0]], rowsv[j],
                                  gsem[j]).wait()
            pltpu.make_async_copy(eh_hbm.at[0], ehv[j], gsem[j]).wait()

        def compute(j):
            rj, ej = rowsv[j], ehv[j]

            def crow(r, cc):
                for q in range(HID // L):
                    sl = pl.ds(q * L, L)
                    rj[r, sl] = jnp.maximum(rj[r, sl] + ej[r, sl], 0.0)
                return cc

            lax.fori_loop(0, CH, crow, 0)

        def issue_sc(j):
            pltpu.async_copy(rowsv[j], agg.at[idxv[j].at[1]], ssem[j],
                             add=True)

        def wait_sc(j):
            pltpu.make_async_copy(rowsv[j], agg.at[idxv[j].at[1]],
                                  ssem[j]).wait()

        zero = jnp.zeros((L,), jnp.float32)

        def zrow(r, cc):
            for q in range(HID // L):
                rowsv[0][r, pl.ds(q * L, L)] = zero
            return cc

        lax.fori_loop(0, CH, zrow, 0)

        for i in range(SLAB // CH):
            pltpu.sync_copy(rowsv[0],
                            agg.at[pl.ds(pl.multiple_of(s * SLAB + i * CH, 8),
                                         CH)])
        plsc.subcore_barrier()

        issue_idx(0, 0)
        issue_idx(1, 1)
        issue_idx(2, 2)
        wait_idx(0)
        issue_ge(0, 0)
        wait_idx(1)
        issue_ge(1, 1)

        def body(g, carry):
            for j in range(SLOTS):
                k = SLOTS * g + j

                @pl.when(k < NCHUNKS)
                def _():
                    wait_ge(j)
                    compute(j)
                    issue_sc(j)

                @pl.when((k >= 1) & (k <= NCHUNKS))
                def _():
                    wait_sc((j + 3) % SLOTS)

                @pl.when(k + 3 < NCHUNKS)
                def _():
                    issue_idx(k + 3, (j + 3) % SLOTS)

                @pl.when(k + 2 < NCHUNKS)
                def _():
                    wait_idx((j + 2) % SLOTS)
                    issue_ge(k + 2, (j + 2) % SLOTS)

            return carry

        lax.fori_loop(0, (NCHUNKS + 2 + SLOTS - 1) // SLOTS, body, 0)
        plsc.subcore_barrier()
        base = pl.multiple_of(s * SLAB, 8)
        pltpu.sync_copy(agg.at[pl.ds(base, SLAB)],
                        out_hbm.at[c, pl.ds(base, SLAB)])

    return kfn(h, eh3d, idx3d)


def _tc_matmul_bias(xm, wm, b2d, bm, relu):
    m, k = xm.shape
    n = wm.shape[1]

    def body(x_ref, w_ref, b_ref, o_ref):
        acc = jnp.dot(x_ref[...], w_ref[...],
                      preferred_element_type=jnp.float32) + b_ref[...]
        o_ref[...] = jnp.maximum(acc, 0.0) if relu else acc

    return pl.pallas_call(
        body,
        grid=(m // bm,),
        in_specs=[
            pl.BlockSpec((bm, k), lambda i: (i, 0)),
            pl.BlockSpec((k, n), lambda i: (0, 0)),
            pl.BlockSpec((1, n), lambda i: (0, 0)),
        ],
        out_specs=pl.BlockSpec((bm, n), lambda i: (i, 0)),
        out_shape=jax.ShapeDtypeStruct((m, n), jnp.float32),
    )(xm, wm, b2d)


def _tc_mlp(agg_p, w1, b1_2d, w2, b2_2d, bm):
    def body(a_ref, w1_ref, b1_ref, w2_ref, b2_ref, o_ref):
        sm = a_ref[0] + a_ref[1]
        z = jnp.maximum(
            jnp.dot(sm, w1_ref[...], preferred_element_type=jnp.float32)
            + b1_ref[...], 0.0)
        o_ref[...] = jnp.maximum(
            jnp.dot(z, w2_ref[...], preferred_element_type=jnp.float32)
            + b2_ref[...], 0.0)

    return pl.pallas_call(
        body,
        grid=(N // bm,),
        in_specs=[
            pl.BlockSpec((NC, bm, HID), lambda i: (0, i, 0)),
            pl.BlockSpec((HID, HID), lambda i: (0, 0)),
            pl.BlockSpec((1, HID), lambda i: (0, 0)),
            pl.BlockSpec((HID, HID), lambda i: (0, 0)),
            pl.BlockSpec((1, HID), lambda i: (0, 0)),
        ],
        out_specs=pl.BlockSpec((bm, HID), lambda i: (i, 0)),
        out_shape=jax.ShapeDtypeStruct((N, HID), jnp.float32),
    )(agg_p, w1, b1_2d, w2, b2_2d)


def _tc_readout(h, w_out, b_out_2d, bv3d, bm):
    nblk = N // bm

    def body(h_ref, w_ref, bo_ref, bv_ref, o_ref):
        i = pl.program_id(0)
        hv = jnp.dot(h_ref[...], w_ref[...],
                     preferred_element_type=jnp.float32,
                     precision=lax.Precision.HIGHEST)
        bv = bv_ref[0, 0, :]
        gi = lax.broadcasted_iota(jnp.int32, (bm, G), 1)
        oh = (bv[:, None] == gi).astype(jnp.float32)
        contrib = lax.dot_general(hv, oh, (((0,), (0,)), ((), ())),
                                  preferred_element_type=jnp.float32,
                                  precision=lax.Precision.HIGHEST)

        @pl.when(i == 0)
        def _():
            o_ref[...] = contrib + bo_ref[...]

        @pl.when(i != 0)
        def _():
            o_ref[...] = o_ref[...] + contrib

    return pl.pallas_call(
        body,
        grid=(nblk,),
        in_specs=[
            pl.BlockSpec((bm, HID), lambda i: (i, 0)),
            pl.BlockSpec((HID, 1), lambda i: (0, 0)),
            pl.BlockSpec((1, 1), lambda i: (0, 0)),
            pl.BlockSpec((1, 1, bm), lambda i: (i, 0, 0)),
        ],
        out_specs=pl.BlockSpec((1, G), lambda i: (0, 0)),
        out_shape=jax.ShapeDtypeStruct((1, G), jnp.float32),
    )(h, w_out, b_out_2d, bv3d)


def kernel(x, edge_attr, W_node, b_node, W_edge, b_edge, W1, b1, W2, b2,
           W_out, b_out, edge_index, batch_vec):
    h = _tc_matmul_bias(x, W_node, b_node.reshape(1, -1), bm=2000, relu=True)
    eh = _tc_matmul_bias(edge_attr, W_edge, b_edge.reshape(1, -1), bm=8000,
                         relu=False)
    eh3d = eh.reshape(E // CH, CH, HID)
    idx3d = edge_index.reshape(2, E // CH, CH).transpose(1, 0, 2)
    for l in range(W1.shape[0]):
        agg_p = _sc_message(h, eh3d, idx3d)
        h = _tc_mlp(agg_p, W1[l], b1[l].reshape(1, -1), W2[l],
                    b2[l].reshape(1, -1), bm=2000)
    pred_t = _tc_readout(h, W_out, b_out.reshape(1, 1),
                         batch_vec.reshape(N // 2000, 1, 2000), bm=2000)
    return pred_t.reshape(G, 1)


# speedup vs baseline: 6.3318x; 1.1163x over previous
import functools

import jax
import jax.numpy as jnp
from jax import lax
from jax.experimental import pallas as pl
from jax.experimental.pallas import tpu as pltpu
from jax.experimental.pallas import tpu_sc as plsc

N = 10000
E = 320000
HID = 128
G = 128
L = 16
NC = 2
NS = 16
NW = NC * NS
EPT = E // NW
CH = 80
NCHUNKS = EPT // CH
SLOTS = 4
PADN = 10240
SLAB = PADN // NS


def _sc_message(h, eh3d, idx3d):
    mesh = plsc.VectorSubcoreMesh(core_axis_name="c", subcore_axis_name="s")

    @functools.partial(
        pl.kernel,
        out_type=jax.ShapeDtypeStruct((NC, PADN, HID), jnp.float32),
        mesh=mesh,
        scratch_types=(
            [pltpu.VMEM((2, CH), jnp.int32) for _ in range(SLOTS)]
            + [pltpu.VMEM((CH, HID), jnp.float32) for _ in range(SLOTS)]
            + [pltpu.SemaphoreType.DMA for _ in range(4 * SLOTS)]
            + [pltpu.VMEM_SHARED((PADN, HID), jnp.float32)]
        ),
    )
    def kfn(h_hbm, eh_hbm, idx_hbm, out_hbm, *refs):
        idxv = refs[0:SLOTS]
        mv = refs[SLOTS:2 * SLOTS]
        isem = refs[2 * SLOTS:3 * SLOTS]
        ehsem = refs[3 * SLOTS:4 * SLOTS]
        gasem = refs[4 * SLOTS:5 * SLOTS]
        ssem = refs[5 * SLOTS:6 * SLOTS]
        agg = refs[6 * SLOTS]

        c = lax.axis_index("c")
        s = lax.axis_index("s")
        w = c * NS + s

        def issue_idx(k, j):
            pltpu.async_copy(idx_hbm.at[w * NCHUNKS + k], idxv[j], isem[j])

        def wait_idx(j):
            pltpu.make_async_copy(idx_hbm.at[0], idxv[j], isem[j]).wait()

        def issue_eh(k, j):
            pltpu.async_copy(eh_hbm.at[w * NCHUNKS + k], mv[j], ehsem[j])

        def wait_eh(j):
            pltpu.make_async_copy(eh_hbm.at[0], mv[j], ehsem[j]).wait()

        def issue_ga(j):
            pltpu.async_copy(h_hbm.at[idxv[j].at[0]], mv[j], gasem[j],
                             add=True)

        def wait_ga(j):
            pltpu.make_async_copy(h_hbm.at[idxv[j].at[0]], mv[j],
                                  gasem[j]).wait()

        def compute(j):
            rj = mv[j]

            def crow(r2, cc):
                for dr in range(2):
                    r = r2 * 2 + dr
                    for q in range(HID // L):
                        sl = pl.ds(q * L, L)
                        rj[r, sl] = jnp.maximum(rj[r, sl], 0.0)
                return cc

            lax.fori_loop(0, CH // 2, crow, 0)

        def issue_sc(j):
            pltpu.async_copy(mv[j], agg.at[idxv[j].at[1]], ssem[j],
                             add=True)

        def wait_sc(j):
            pltpu.make_async_copy(mv[j], agg.at[idxv[j].at[1]],
                                  ssem[j]).wait()

        zero = jnp.zeros((L,), jnp.float32)

        def zrow(r, cc):
            for q in range(HID // L):
                mv[0][r, pl.ds(q * L, L)] = zero
            return cc

        lax.fori_loop(0, 40, zrow, 0)

        for i in range(SLAB // 40):
            pltpu.sync_copy(mv[0].at[pl.ds(0, 40)],
                            agg.at[pl.ds(pl.multiple_of(s * SLAB + i * 40, 8),
                                         40)])
        plsc.subcore_barrier()

        issue_idx(0, 0)
        issue_idx(1, 1)
        issue_idx(2, 2)
        issue_eh(0, 0)
        issue_eh(1, 1)
        issue_eh(2, 2)
        wait_idx(0)
        wait_eh(0)
        issue_ga(0)
        wait_idx(1)
        wait_eh(1)
        issue_ga(1)

        def body(g, carry):
            for j in range(SLOTS):
                k = SLOTS * g + j

                @pl.when(k < NCHUNKS)
                def _():
                    wait_ga(j)
                    compute(j)
                    issue_sc(j)

                @pl.when((k >= 1) & (k <= NCHUNKS))
                def _():
                    wait_sc((j + 3) % SLOTS)

                @pl.when(k + 3 < NCHUNKS)
                def _():
                    issue_idx(k + 3, (j + 3) % SLOTS)
                    issue_eh(k + 3, (j + 3) % SLOTS)

                @pl.when(k + 2 < NCHUNKS)
                def _():
                    wait_idx((j + 2) % SLOTS)
                    wait_eh((j + 2) % SLOTS)
                    issue_ga((j + 2) % SLOTS)

            return carry

        lax.fori_loop(0, (NCHUNKS + 2 + SLOTS - 1) // SLOTS, body, 0)
        plsc.subcore_barrier()
        base = pl.multiple_of(s * SLAB, 8)
        pltpu.sync_copy(agg.at[pl.ds(base, SLAB)],
                        out_hbm.at[c, pl.ds(base, SLAB)])

    return kfn(h, eh3d, idx3d)


def _tc_matmul_bias(xm, wm, b2d, bm, relu):
    m, k = xm.shape
    n = wm.shape[1]

    def body(x_ref, w_ref, b_ref, o_ref):
        acc = jnp.dot(x_ref[...], w_ref[...],
                      preferred_element_type=jnp.float32) + b_ref[...]
        o_ref[...] = jnp.maximum(acc, 0.0) if relu else acc

    return pl.pallas_call(
        body,
        grid=(m // bm,),
        in_specs=[
            pl.BlockSpec((bm, k), lambda i: (i, 0)),
            pl.BlockSpec((k, n), lambda i: (0, 0)),
            pl.BlockSpec((1, n), lambda i: (0, 0)),
        ],
        out_specs=pl.BlockSpec((bm, n), lambda i: (i, 0)),
        out_shape=jax.ShapeDtypeStruct((m, n), jnp.float32),
    )(xm, wm, b2d)


def _tc_edge_embed(xm, wm, b2d, bm):
    m, k = xm.shape
    n = wm.shape[1]

    def body(x_ref, w_ref, b_ref, o_ref):
        acc = jnp.dot(x_ref[...], w_ref[...],
                      preferred_element_type=jnp.float32) + b_ref[...]
        o_ref[...] = acc.reshape(bm // CH, CH, n)

    return pl.pallas_call(
        body,
        grid=(m // bm,),
        in_specs=[
            pl.BlockSpec((bm, k), lambda i: (i, 0)),
            pl.BlockSpec((k, n), lambda i: (0, 0)),
            pl.BlockSpec((1, n), lambda i: (0, 0)),
        ],
        out_specs=pl.BlockSpec((bm // CH, CH, n), lambda i: (i, 0, 0)),
        out_shape=jax.ShapeDtypeStruct((m // CH, CH, n), jnp.float32),
    )(xm, wm, b2d)


def _tc_mlp(agg_p, w1, b1_2d, w2, b2_2d, bm):
    def body(a_ref, w1_ref, b1_ref, w2_ref, b2_ref, o_ref):
        sm = a_ref[0] + a_ref[1]
        z = jnp.maximum(
            jnp.dot(sm, w1_ref[...], preferred_element_type=jnp.float32)
            + b1_ref[...], 0.0)
        o_ref[...] = jnp.maximum(
            jnp.dot(z, w2_ref[...], preferred_element_type=jnp.float32)
            + b2_ref[...], 0.0)

    return pl.pallas_call(
        body,
        grid=(N // bm,),
        in_specs=[
            pl.BlockSpec((NC, bm, HID), lambda i: (0, i, 0)),
            pl.BlockSpec((HID, HID), lambda i: (0, 0)),
            pl.BlockSpec((1, HID), lambda i: (0, 0)),
            pl.BlockSpec((HID, HID), lambda i: (0, 0)),
            pl.BlockSpec((1, HID), lambda i: (0, 0)),
        ],
        out_specs=pl.BlockSpec((bm, HID), lambda i: (i, 0)),
        out_shape=jax.ShapeDtypeStruct((N, HID), jnp.float32),
    )(agg_p, w1, b1_2d, w2, b2_2d)


def _tc_readout(h, w_out, b_out_2d, bv3d, bm):
    nblk = N // bm

    def body(h_ref, w_ref, bo_ref, bv_ref, o_ref):
        i = pl.program_id(0)
        hv = jnp.dot(h_ref[...], w_ref[...],
                     preferred_element_type=jnp.float32,
                     precision=lax.Precision.HIGHEST)
        bv = bv_ref[0, 0, :]
        gi = lax.broadcasted_iota(jnp.int32, (bm, G), 1)
        oh = (bv[:, None] == gi).astype(jnp.float32)
        contrib = lax.dot_general(hv, oh, (((0,), (0,)), ((), ())),
                                  preferred_element_type=jnp.float32,
                                  precision=lax.Precision.HIGHEST)

        @pl.when(i == 0)
        def _():
            o_ref[...] = contrib + bo_ref[...]

        @pl.when(i != 0)
        def _():
            o_ref[...] = o_ref[...] + contrib

    return pl.pallas_call(
        body,
        grid=(nblk,),
        in_specs=[
            pl.BlockSpec((bm, HID), lambda i: (i, 0)),
            pl.BlockSpec((HID, 1), lambda i: (0, 0)),
            pl.BlockSpec((1, 1), lambda i: (0, 0)),
            pl.BlockSpec((1, 1, bm), lambda i: (i, 0, 0)),
        ],
        out_specs=pl.BlockSpec((1, G), lambda i: (0, 0)),
        out_shape=jax.ShapeDtypeStruct((1, G), jnp.float32),
    )(h, w_out, b_out_2d, bv3d)


def kernel(x, edge_attr, W_node, b_node, W_edge, b_edge, W1, b1, W2, b2,
           W_out, b_out, edge_index, batch_vec):
    h = _tc_matmul_bias(x, W_node, b_node.reshape(1, -1), bm=2000, relu=True)
    eh3d = _tc_edge_embed(edge_attr, W_edge, b_edge.reshape(1, -1), bm=8000)
    idx3d = edge_index.reshape(2, E // CH, CH).transpose(1, 0, 2)
    for l in range(W1.shape[0]):
        agg_p = _sc_message(h, eh3d, idx3d)
        h = _tc_mlp(agg_p, W1[l], b1[l].reshape(1, -1), W2[l],
                    b2[l].reshape(1, -1), bm=2000)
    pred_t = _tc_readout(h, W_out, b_out.reshape(1, 1),
                         batch_vec.reshape(N // 2000, 1, 2000), bm=2000)
    return pred_t.reshape(G, 1)
